# baseline (speedup 1.0000x reference)
import functools
import math

import jax
import jax.numpy as jnp
from jax import lax
from jax.experimental import pallas as pl
from jax.experimental.pallas import tpu as pltpu
from jax.experimental.pallas import tpu_sc as plsc

D = 128
SCALE = math.sqrt(float(D))
LANES = 16
NC, NS = 2, 16
NW = NC * NS

B_TOTAL = 1024 * 200
PER_W = B_TOTAL // NW
CHUNK = 128
NCH = PER_W // CHUNK
NPAIR = NCH // 2


def _scale_chunk(src, dst):
    def row(r, c):
        for s in range(D // LANES):
            sl = pl.ds(s * LANES, LANES)
            dst[r, sl] = src[r, sl] * SCALE
        return c

    lax.fori_loop(0, CHUNK, row, 0, unroll=2)


def _emb_body(seqs_hbm, table_hbm, out_hbm,
              idx_all, g0, g1, s0, s1,
              semg0, semg1, sems0, sems1):
    wid = lax.axis_index("s") * NC + lax.axis_index("c")
    base = wid * PER_W
    gbuf = (g0, g1)
    sbuf = (s0, s1)
    semg = (semg0, semg1)
    sems = (sems0, sems1)

    def idx_ref(k):
        return idx_all.at[pl.ds(k * CHUNK, CHUNK)]

    def gather(k, b):
        return pltpu.make_async_copy(table_hbm.at[idx_ref(k)], gbuf[b], semg[b])

    def scatter(k, b):
        return pltpu.make_async_copy(
            sbuf[b], out_hbm.at[pl.ds(base + k * CHUNK, CHUNK)], sems[b])

    pltpu.sync_copy(seqs_hbm.at[pl.ds(base, PER_W)], idx_all)

    gather(0, 0).start()
    gather(1, 1).start()

    def pair(j, carry):
        for b in range(2):
            k = 2 * j + b
            gather(k, b).wait()

            @pl.when(j >= 1)
            def _():
                scatter(k - 2, b).wait()

            _scale_chunk(gbuf[b], sbuf[b])

            @pl.when(j < NPAIR - 1)
            def _():
                gather(k + 2, b).start()

            scatter(k, b).start()
        return carry

    lax.fori_loop(0, NPAIR, pair, 0)

    scatter(NCH - 2, 0).wait()
    scatter(NCH - 1, 1).wait()


_emb_lookup = functools.partial(
    pl.kernel,
    out_type=jax.ShapeDtypeStruct((B_TOTAL, D), jnp.float32),
    mesh=plsc.VectorSubcoreMesh(core_axis_name="c", subcore_axis_name="s"),
    scratch_types=[
        pltpu.VMEM((PER_W,), jnp.int32),
        pltpu.VMEM((CHUNK, D), jnp.float32),
        pltpu.VMEM((CHUNK, D), jnp.float32),
        pltpu.VMEM((CHUNK, D), jnp.float32),
        pltpu.VMEM((CHUNK, D), jnp.float32),
        pltpu.SemaphoreType.DMA,
        pltpu.SemaphoreType.DMA,
        pltpu.SemaphoreType.DMA,
        pltpu.SemaphoreType.DMA,
    ],
)(_emb_body)


def _mask_body(lens_ref, out_ref):
    pos = lax.broadcasted_iota(jnp.int32, out_ref.shape, 1)
    valid = pos < lens_ref[:]
    out_ref[:] = jnp.where(valid, jnp.float32(0.0), jnp.float32(-jnp.inf))


def kernel(seqs, seq_lens, embed_table):
    bsz, seq_len = seqs.shape
    flat = seqs.reshape(-1).astype(jnp.int32)
    emb = _emb_lookup(flat, embed_table)
    mask = pl.pallas_call(
        _mask_body,
        out_shape=jax.ShapeDtypeStruct((bsz, seq_len), jnp.float32),
    )(seq_lens.reshape(bsz, 1))
    return emb.reshape(bsz, seq_len, D), mask


# speedup vs baseline: 1.6632x; 1.6632x over previous
import functools
import math

import jax
import jax.numpy as jnp
from jax import lax
from jax.experimental import pallas as pl
from jax.experimental.pallas import tpu as pltpu
from jax.experimental.pallas import tpu_sc as plsc

D = 128
SCALE = math.sqrt(float(D))
LANES = 16
NC, NS = 2, 16
NW = NC * NS

B_TOTAL = 1024 * 200
PER_W = B_TOTAL // NW
CHUNK = 128
NCH = PER_W // CHUNK
NPAIR = NCH // 2


def _scale_chunk(buf):
    def row(r, c):
        for s in range(D // LANES):
            sl = pl.ds(s * LANES, LANES)
            buf[r, sl] = buf[r, sl] * SCALE
        return c

    lax.fori_loop(0, CHUNK, row, 0)


def _emb_body(seqs_hbm, table_hbm, out_hbm,
              i0, i1, g0, g1,
              semg0, semg1, sems0, sems1):
    wid = lax.axis_index("s") * NC + lax.axis_index("c")
    base = wid * PER_W
    ibuf = (i0, i1)
    gbuf = (g0, g1)
    semg = (semg0, semg1)
    sems = (sems0, sems1)

    def gather(b):
        return pltpu.make_async_copy(table_hbm.at[ibuf[b]], gbuf[b], semg[b])

    def scatter(k, b):
        return pltpu.make_async_copy(
            gbuf[b], out_hbm.at[pl.ds(base + k * CHUNK, CHUNK)], sems[b])

    def pair(j, carry):
        for b in range(2):
            k = 2 * j + b
            pltpu.sync_copy(seqs_hbm.at[pl.ds(base + k * CHUNK, CHUNK)], ibuf[b])

            @pl.when(j >= 1)
            def _():
                scatter(k - 2, b).wait()

            gather(b).start()
            gather(b).wait()
            _scale_chunk(gbuf[b])
            scatter(k, b).start()
        return carry

    lax.fori_loop(0, NPAIR, pair, 0)

    scatter(NCH - 2, 0).wait()
    scatter(NCH - 1, 1).wait()


_emb_lookup = functools.partial(
    pl.kernel,
    out_type=jax.ShapeDtypeStruct((B_TOTAL, D), jnp.float32),
    mesh=plsc.VectorSubcoreMesh(core_axis_name="c", subcore_axis_name="s"),
    scratch_types=[
        pltpu.VMEM((CHUNK,), jnp.int32),
        pltpu.VMEM((CHUNK,), jnp.int32),
        pltpu.VMEM((CHUNK, D), jnp.float32),
        pltpu.VMEM((CHUNK, D), jnp.float32),
        pltpu.SemaphoreType.DMA,
        pltpu.SemaphoreType.DMA,
        pltpu.SemaphoreType.DMA,
        pltpu.SemaphoreType.DMA,
    ],
)(_emb_body)


def _mask_body(lens_ref, out_ref):
    pos = lax.broadcasted_iota(jnp.int32, out_ref.shape, 1)
    valid = pos < lens_ref[:]
    out_ref[:] = jnp.where(valid, jnp.float32(0.0), jnp.float32(-jnp.inf))


def kernel(seqs, seq_lens, embed_table):
    bsz, seq_len = seqs.shape
    flat = seqs.reshape(-1).astype(jnp.int32)
    emb = _emb_lookup(flat, embed_table)
    mask = pl.pallas_call(
        _mask_body,
        out_shape=jax.ShapeDtypeStruct((bsz, seq_len), jnp.float32),
    )(seq_lens.reshape(bsz, 1))
    return emb.reshape(bsz, seq_len, D), mask


# speedup vs baseline: 2.3994x; 1.4426x over previous
import functools
import math

import jax
import jax.numpy as jnp
from jax import lax
from jax.experimental import pallas as pl
from jax.experimental.pallas import tpu as pltpu
from jax.experimental.pallas import tpu_sc as plsc

D = 128
SCALE = math.sqrt(float(D))
LANES = 16
NC, NS = 2, 16
NW = NC * NS

B_TOTAL = 1024 * 200
PER_W = B_TOTAL // NW
CHUNK = 128
NCH = PER_W // CHUNK
NPAIR = NCH // 2


def _scale_chunk(src, dst):
    def row(r, c):
        for s in range(D // LANES):
            sl = pl.ds(s * LANES, LANES)
            dst[r, sl] = src[r, sl] * SCALE
        return c

    lax.fori_loop(0, CHUNK, row, 0)


def _emb_body(seqs_hbm, table_hbm, out_hbm,
              i0, i1, g0, g1, s0, s1,
              semi0, semi1, semg0, semg1, sems0, sems1):
    wid = lax.axis_index("s") * NC + lax.axis_index("c")
    base = wid * PER_W
    ibuf = (i0, i1)
    gbuf = (g0, g1)
    sbuf = (s0, s1)
    semi = (semi0, semi1)
    semg = (semg0, semg1)
    sems = (sems0, sems1)

    def idxload(k, b):
        return pltpu.make_async_copy(
            seqs_hbm.at[pl.ds(base + k * CHUNK, CHUNK)], ibuf[b], semi[b])

    def gather(b):
        return pltpu.make_async_copy(table_hbm.at[ibuf[b]], gbuf[b], semg[b])

    def scatter(k, b):
        return pltpu.make_async_copy(
            sbuf[b], out_hbm.at[pl.ds(base + k * CHUNK, CHUNK)], sems[b])

    pltpu.sync_copy(seqs_hbm.at[pl.ds(base, CHUNK)], i0)
    gather(0).start()
    idxload(1, 1).start()

    def pair(j, carry):
        for b in range(2):
            k = 2 * j + b
            gather(b).wait()

            @pl.when(j < NPAIR - 1)
            def _():
                idxload(k + 2, b).start()

            if b == 0:
                idxload(k + 1, 1).wait()
                gather(1).start()
            else:
                @pl.when(j < NPAIR - 1)
                def _():
                    idxload(k + 1, 0).wait()
                    gather(0).start()

            @pl.when(j >= 1)
            def _():
                scatter(k - 2, b).wait()

            _scale_chunk(gbuf[b], sbuf[b])
            scatter(k, b).start()
        return carry

    lax.fori_loop(0, NPAIR, pair, 0)

    scatter(NCH - 2, 0).wait()
    scatter(NCH - 1, 1).wait()


_emb_lookup = functools.partial(
    pl.kernel,
    out_type=jax.ShapeDtypeStruct((B_TOTAL, D), jnp.float32),
    mesh=plsc.VectorSubcoreMesh(core_axis_name="c", subcore_axis_name="s"),
    scratch_types=[
        pltpu.VMEM((CHUNK,), jnp.int32),
        pltpu.VMEM((CHUNK,), jnp.int32),
        pltpu.VMEM((CHUNK, D), jnp.float32),
        pltpu.VMEM((CHUNK, D), jnp.float32),
        pltpu.VMEM((CHUNK, D), jnp.float32),
        pltpu.VMEM((CHUNK, D), jnp.float32),
        pltpu.SemaphoreType.DMA,
        pltpu.SemaphoreType.DMA,
        pltpu.SemaphoreType.DMA,
        pltpu.SemaphoreType.DMA,
        pltpu.SemaphoreType.DMA,
        pltpu.SemaphoreType.DMA,
    ],
)(_emb_body)


def _mask_body(lens_ref, out_ref):
    pos = lax.broadcasted_iota(jnp.int32, out_ref.shape, 1)
    valid = pos < lens_ref[:]
    out_ref[:] = jnp.where(valid, jnp.float32(0.0), jnp.float32(-jnp.inf))


def kernel(seqs, seq_lens, embed_table):
    bsz, seq_len = seqs.shape
    flat = seqs.reshape(-1).astype(jnp.int32)
    emb = _emb_lookup(flat, embed_table)
    mask = pl.pallas_call(
        _mask_body,
        out_shape=jax.ShapeDtypeStruct((bsz, seq_len), jnp.float32),
    )(seq_lens.reshape(bsz, 1))
    return emb.reshape(bsz, seq_len, D), mask
